# baseline (speedup 1.0000x reference)
import functools

import jax
import jax.numpy as jnp
from jax import lax
from jax.experimental import pallas as pl
from jax.experimental.pallas import tpu as pltpu
from jax.experimental.pallas import tpu_sc as plsc

N_NODES = 10000
N_P = 10240
E = 320000
NC, NS = 2, 16
NW = NC * NS
EB = 128
ROWS_T = 2560
E_P = ROWS_T * EB
D = 64
KPROP = 10
ALPHA = 0.1
ROWS_PER_TILE = N_P // NS

A_ROWS = 80


def _sc_scatter_body(q_hbm, src_hbm, dst_hbm, agg_hbm,
                     sidx_v, didx_v, rows_v, zbuf_v,
                     gsem0, gsem1, ssem0, ssem1, agg_sh):
    cid = lax.axis_index("c")
    sid = lax.axis_index("s")

    @pl.when(cid == 0)
    def _():
        zv = jnp.zeros((16,), jnp.float32)

        def zrow(r, carry):
            for kk in range(D // 16):
                zbuf_v[r, pl.ds(kk * 16, 16)] = zv
            return carry

        lax.fori_loop(0, 128, zrow, 0, unroll=False)
        zbase = sid * ROWS_PER_TILE
        for i in range(ROWS_PER_TILE // 128):
            pltpu.sync_copy(zbuf_v, agg_sh.at[pl.ds(zbase + i * 128, 128)])

        row0 = sid * A_ROWS
        pltpu.sync_copy(src_hbm.at[pl.ds(row0, A_ROWS)], sidx_v)
        pltpu.sync_copy(dst_hbm.at[pl.ds(row0, A_ROWS)], didx_v)

        plsc.subcore_barrier()

        def gather(g, bank, sem):
            j = jnp.minimum(g, A_ROWS - 1)
            pltpu.async_copy(q_hbm.at[sidx_v.at[j]], rows_v.at[bank], sem)

        def scatter(g, bank, sem):
            pltpu.async_copy(rows_v.at[bank], agg_sh.at[didx_v.at[g]],
                             sem, add=True)

        def drain(sem):
            pltpu.make_async_copy(q_hbm.at[pl.ds(0, EB)], rows_v.at[0],
                                  sem).wait()

        gather(0, 0, gsem0)

        def pair(t, carry):
            g0 = 2 * t
            drain(gsem0)
            scatter(g0, 0, ssem0)
            @pl.when(t > 0)
            def _():
                drain(ssem1)
            gather(g0 + 1, 1, gsem1)
            drain(gsem1)
            scatter(g0 + 1, 1, ssem1)
            drain(ssem0)
            gather(g0 + 2, 0, gsem0)
            return carry

        lax.fori_loop(0, A_ROWS // 2, pair, 0, unroll=False)
        drain(gsem0)
        drain(ssem1)

        plsc.subcore_barrier()
        pltpu.sync_copy(agg_sh.at[pl.ds(zbase, ROWS_PER_TILE)],
                        agg_hbm.at[pl.ds(zbase, ROWS_PER_TILE)])


def _sc_scatter(q, src3, dst3):
    mesh = plsc.VectorSubcoreMesh(core_axis_name="c", subcore_axis_name="s",
                                  num_cores=NC, num_subcores=NS)
    f = pl.kernel(
        _sc_scatter_body,
        out_type=jax.ShapeDtypeStruct((N_P, D), jnp.float32),
        mesh=mesh,
        scratch_types=[
            pltpu.VMEM((A_ROWS, EB), jnp.int32),
            pltpu.VMEM((A_ROWS, EB), jnp.int32),
            pltpu.VMEM((2, EB, D), jnp.float32),
            pltpu.VMEM((128, D), jnp.float32),
            pltpu.SemaphoreType.DMA,
            pltpu.SemaphoreType.DMA,
            pltpu.SemaphoreType.DMA,
            pltpu.SemaphoreType.DMA,
            pltpu.VMEM_SHARED((N_P, D), jnp.float32),
        ],
        compiler_params=pltpu.CompilerParams(use_tc_tiling_on_sc=False),
    )
    return f(q, src3, dst3)


def _tc_prep_body(agga_ref, aggb_ref, dinv_ref, cf_ref, dsqrt_ref):
    indeg = agga_ref[:, 0:1] + aggb_ref[:, 0:1]
    row = lax.broadcasted_iota(jnp.int32, (N_P, 1), 0)
    valid = row < N_NODES
    deg = jnp.where(valid, indeg + 1.0, 1.0)
    dinv = jax.lax.rsqrt(deg)
    dinv = jnp.where(valid, dinv, 0.0)
    dinv_ref[...] = dinv
    cf_ref[...] = (1.0 - ALPHA) * dinv * dinv
    dsqrt_ref[...] = jnp.where(valid, jnp.sqrt(deg), 0.0)


def _tc_prep(agga, aggb):
    return pl.pallas_call(
        _tc_prep_body,
        out_shape=[jax.ShapeDtypeStruct((N_P, 1), jnp.float32)] * 3,
    )(agga, aggb)


def _tc_in_body(x_ref, w_ref, b_ref, dinv_ref, q_ref):
    h = jnp.dot(x_ref[...], w_ref[...].T,
                preferred_element_type=jnp.float32) + b_ref[...]
    q_ref[...] = dinv_ref[...] * h


def _tc_in(x, W1, b1, dinv):
    return pl.pallas_call(
        _tc_in_body,
        out_shape=jax.ShapeDtypeStruct((N_P, D), jnp.float32),
    )(x, W1, b1[None, :], dinv)


def _tc_scale_body(agga_ref, aggb_ref, q_ref, r0_ref, cf_ref, out_ref):
    s = agga_ref[...] + aggb_ref[...] + q_ref[...]
    out_ref[...] = cf_ref[...] * s + ALPHA * r0_ref[...]


def _tc_scale(agga, aggb, q, r0, cf):
    return pl.pallas_call(
        _tc_scale_body,
        out_shape=jax.ShapeDtypeStruct((N_P, D), jnp.float32),
    )(agga, aggb, q, r0, cf)


def _tc_mid_body(q_ref, dsqrt_ref, dinv_ref, w_ref, b_ref, out_ref):
    h = jax.nn.relu(dsqrt_ref[...] * q_ref[...])
    h = jnp.dot(h, w_ref[...].T, preferred_element_type=jnp.float32) + b_ref[...]
    out_ref[...] = dinv_ref[...] * h


def _tc_mid(q, dsqrt, dinv, W, b):
    return pl.pallas_call(
        _tc_mid_body,
        out_shape=jax.ShapeDtypeStruct((N_P, D), jnp.float32),
    )(q, dsqrt, dinv, W, b[None, :])


def _tc_head_body(q_ref, dsqrt_ref, w_ref, b_ref, out_ref):
    h = jax.nn.relu(dsqrt_ref[...] * q_ref[...])
    h = jnp.dot(h, w_ref[...].T, preferred_element_type=jnp.float32) + b_ref[...]
    m = jnp.max(h, axis=1, keepdims=True)
    e = jnp.exp(h - m)
    lse = jnp.log(jnp.sum(e, axis=1, keepdims=True)) + m
    out_ref[...] = h - lse


def _tc_head(q, dsqrt, Wc, bc):
    return pl.pallas_call(
        _tc_head_body,
        out_shape=jax.ShapeDtypeStruct((N_P, 8), jnp.float32),
    )(q, dsqrt, Wc, bc[None, :])


def kernel(x, edge_index, W1, b1, W2, b2, W3, b3, Wc, bc):
    src = edge_index[0].astype(jnp.int32)
    dst = edge_index[1].astype(jnp.int32)
    pad_e = ROWS_T * EB - E
    fill = jnp.full((pad_e,), N_P - 1, jnp.int32)
    src3 = jnp.concatenate([src, fill]).reshape(ROWS_T, EB)
    dst3 = jnp.concatenate([dst, fill]).reshape(ROWS_T, EB)

    half = ROWS_T // 2
    srcA, srcB = src3[:half], src3[half:]
    dstA, dstB = dst3[:half], dst3[half:]

    xp = jnp.pad(x, ((0, N_P - N_NODES), (0, 0)))
    ones = jnp.ones((N_P, D), jnp.float32)

    dinv, cf, dsqrt = _tc_prep(_sc_scatter(ones, srcA, dstA),
                               _sc_scatter(ones, srcB, dstB))

    q = _tc_in(xp, W1, b1, dinv)
    for blk, (W, b) in enumerate([(W2, b2), (W3, b3), (None, None)]):
        r0 = q
        for _ in range(KPROP):
            agga = _sc_scatter(q, srcA, dstA)
            aggb = _sc_scatter(q, srcB, dstB)
            q = _tc_scale(agga, aggb, q, r0, cf)
        if W is not None:
            q = _tc_mid(q, dsqrt, dinv, W, b)
    out = _tc_head(q, dsqrt, Wc, bc)
    return out[:N_NODES]


# speedup vs baseline: 1.6126x; 1.6126x over previous
import functools

import jax
import jax.numpy as jnp
from jax import lax
from jax.experimental import pallas as pl
from jax.experimental.pallas import tpu as pltpu
from jax.experimental.pallas import tpu_sc as plsc

N_NODES = 10000
N_P = 10240
E = 320000
NC, NS = 2, 16
NW = NC * NS
EB = 256
ROWS_T = 1280
E_P = ROWS_T * EB
D = 64
KPROP = 10
ALPHA = 0.1
ROWS_PER_TILE = N_P // NS

A_ROWS = 70
B_ROWS = 10


def _sc_scatter_body(q_hbm, src_hbm, dst_hbm, agg_hbm,
                     sidx_v, didx_v, rows_v, zbuf_v,
                     gsem0, gsem1, ssem0, ssem1, agg_sh):
    cid = lax.axis_index("c")
    sid = lax.axis_index("s")

    zv = jnp.zeros((16,), jnp.float32)

    def zrow(r, carry):
        for kk in range(D // 16):
            zbuf_v[r, pl.ds(kk * 16, 16)] = zv
        return carry

    lax.fori_loop(0, 128, zrow, 0, unroll=False)
    zbase = sid * ROWS_PER_TILE
    for i in range(ROWS_PER_TILE // 128):
        pltpu.sync_copy(zbuf_v, agg_sh.at[pl.ds(zbase + i * 128, 128)])

    row0 = jnp.where(cid == 0, sid * A_ROWS, NS * A_ROWS + sid * B_ROWS)
    nrows = jnp.where(cid == 0, A_ROWS, B_ROWS)

    @pl.when(cid == 0)
    def _():
        pltpu.sync_copy(src_hbm.at[pl.ds(row0, A_ROWS)], sidx_v)
        pltpu.sync_copy(dst_hbm.at[pl.ds(row0, A_ROWS)], didx_v)

    @pl.when(cid == 1)
    def _():
        pltpu.sync_copy(src_hbm.at[pl.ds(row0, B_ROWS)],
                        sidx_v.at[pl.ds(0, B_ROWS)])
        pltpu.sync_copy(dst_hbm.at[pl.ds(row0, B_ROWS)],
                        didx_v.at[pl.ds(0, B_ROWS)])

    plsc.subcore_barrier()

    def gather(g, bank, sem):
        j = jnp.minimum(g, nrows - 1)
        pltpu.async_copy(q_hbm.at[sidx_v.at[j]], rows_v.at[bank], sem)

    def scatter(g, bank, sem):
        pltpu.async_copy(rows_v.at[bank], agg_sh.at[didx_v.at[g]],
                         sem, add=True)

    def drain(sem):
        pltpu.make_async_copy(q_hbm.at[pl.ds(0, EB)], rows_v.at[0], sem).wait()

    gather(0, 0, gsem0)

    def pair(t, carry):
        g0 = 2 * t
        drain(gsem0)
        scatter(g0, 0, ssem0)
        @pl.when(t > 0)
        def _():
            drain(ssem1)
        gather(g0 + 1, 1, gsem1)
        drain(gsem1)
        scatter(g0 + 1, 1, ssem1)
        drain(ssem0)
        gather(g0 + 2, 0, gsem0)
        return carry

    lax.fori_loop(0, nrows // 2, pair, 0, unroll=False)
    drain(gsem0)
    drain(ssem1)

    plsc.subcore_barrier()
    obase = cid * N_P + sid * ROWS_PER_TILE
    pltpu.sync_copy(agg_sh.at[pl.ds(zbase, ROWS_PER_TILE)],
                    agg_hbm.at[pl.ds(obase, ROWS_PER_TILE)])


def _sc_scatter(q, src3, dst3):
    mesh = plsc.VectorSubcoreMesh(core_axis_name="c", subcore_axis_name="s",
                                  num_cores=NC, num_subcores=NS)
    f = pl.kernel(
        _sc_scatter_body,
        out_type=jax.ShapeDtypeStruct((NC * N_P, D), jnp.float32),
        mesh=mesh,
        scratch_types=[
            pltpu.VMEM((A_ROWS, EB), jnp.int32),
            pltpu.VMEM((A_ROWS, EB), jnp.int32),
            pltpu.VMEM((2, EB, D), jnp.float32),
            pltpu.VMEM((128, D), jnp.float32),
            pltpu.SemaphoreType.DMA,
            pltpu.SemaphoreType.DMA,
            pltpu.SemaphoreType.DMA,
            pltpu.SemaphoreType.DMA,
            pltpu.VMEM_SHARED((N_P, D), jnp.float32),
        ],
        compiler_params=pltpu.CompilerParams(use_tc_tiling_on_sc=False),
    )
    return f(q, src3, dst3)


def _tc_prep_body(agg_ref, dinv_ref, cf_ref, dsqrt_ref):
    indeg = agg_ref[0:N_P, 0:1] + agg_ref[N_P:2 * N_P, 0:1]
    row = lax.broadcasted_iota(jnp.int32, (N_P, 1), 0)
    valid = row < N_NODES
    deg = jnp.where(valid, indeg + 1.0, 1.0)
    dinv = jax.lax.rsqrt(deg)
    dinv = jnp.where(valid, dinv, 0.0)
    dinv_ref[...] = dinv
    cf_ref[...] = (1.0 - ALPHA) * dinv * dinv
    dsqrt_ref[...] = jnp.where(valid, jnp.sqrt(deg), 0.0)


def _tc_prep(agg):
    return pl.pallas_call(
        _tc_prep_body,
        out_shape=[jax.ShapeDtypeStruct((N_P, 1), jnp.float32)] * 3,
    )(agg)


def _tc_in_body(x_ref, w_ref, b_ref, dinv_ref, q_ref):
    h = jnp.dot(x_ref[...], w_ref[...].T,
                preferred_element_type=jnp.float32) + b_ref[...]
    q_ref[...] = dinv_ref[...] * h


def _tc_in(x, W1, b1, dinv):
    return pl.pallas_call(
        _tc_in_body,
        out_shape=jax.ShapeDtypeStruct((N_P, D), jnp.float32),
    )(x, W1, b1[None, :], dinv)


def _tc_scale_body(agg_ref, q_ref, r0_ref, cf_ref, out_ref):
    s = agg_ref[0:N_P, :] + agg_ref[N_P:2 * N_P, :] + q_ref[...]
    out_ref[...] = cf_ref[...] * s + ALPHA * r0_ref[...]


def _tc_scale(agg, q, r0, cf):
    return pl.pallas_call(
        _tc_scale_body,
        out_shape=jax.ShapeDtypeStruct((N_P, D), jnp.float32),
    )(agg, q, r0, cf)


def _tc_mid_body(q_ref, dsqrt_ref, dinv_ref, w_ref, b_ref, out_ref):
    h = jax.nn.relu(dsqrt_ref[...] * q_ref[...])
    h = jnp.dot(h, w_ref[...].T, preferred_element_type=jnp.float32) + b_ref[...]
    out_ref[...] = dinv_ref[...] * h


def _tc_mid(q, dsqrt, dinv, W, b):
    return pl.pallas_call(
        _tc_mid_body,
        out_shape=jax.ShapeDtypeStruct((N_P, D), jnp.float32),
    )(q, dsqrt, dinv, W, b[None, :])


def _tc_head_body(q_ref, dsqrt_ref, w_ref, b_ref, out_ref):
    h = jax.nn.relu(dsqrt_ref[...] * q_ref[...])
    h = jnp.dot(h, w_ref[...].T, preferred_element_type=jnp.float32) + b_ref[...]
    m = jnp.max(h, axis=1, keepdims=True)
    e = jnp.exp(h - m)
    lse = jnp.log(jnp.sum(e, axis=1, keepdims=True)) + m
    out_ref[...] = h - lse


def _tc_head(q, dsqrt, Wc, bc):
    return pl.pallas_call(
        _tc_head_body,
        out_shape=jax.ShapeDtypeStruct((N_P, 8), jnp.float32),
    )(q, dsqrt, Wc, bc[None, :])


def kernel(x, edge_index, W1, b1, W2, b2, W3, b3, Wc, bc):
    src = edge_index[0].astype(jnp.int32)
    dst = edge_index[1].astype(jnp.int32)
    pad_e = (ROWS_T + A_ROWS) * EB - E
    fill = jnp.full((pad_e,), N_P - 1, jnp.int32)
    src3 = jnp.concatenate([src, fill]).reshape(ROWS_T + A_ROWS, EB)
    dst3 = jnp.concatenate([dst, fill]).reshape(ROWS_T + A_ROWS, EB)

    xp = jnp.pad(x, ((0, N_P - N_NODES), (0, 0)))
    ones = jnp.ones((N_P, D), jnp.float32)

    aggdeg = _sc_scatter(ones, src3, dst3)
    dinv, cf, dsqrt = _tc_prep(aggdeg)

    q = _tc_in(xp, W1, b1, dinv)
    for blk, (W, b) in enumerate([(W2, b2), (W3, b3), (None, None)]):
        r0 = q
        for _ in range(KPROP):
            agg = _sc_scatter(q, src3, dst3)
            q = _tc_scale(agg, q, r0, cf)
        if W is not None:
            q = _tc_mid(q, dsqrt, dinv, W, b)
    out = _tc_head(q, dsqrt, Wc, bc)
    return out[:N_NODES]
